# Initial kernel scaffold; baseline (speedup 1.0000x reference)
import jax
import jax.numpy as jnp
from jax.experimental import pallas as pl


def kernel(x, W1, b1, W2, b2, gate_log_temp, gate_eps_logit):
    raise NotImplementedError("write your pallas kernel here")


# speedup vs baseline: 1.2209x; 1.2209x over previous
import functools

import jax
import jax.numpy as jnp
from jax.experimental import pallas as pl
from jax.experimental.pallas import tpu as pltpu

B, L, H_DIM = 2, 4096, 2048
NH, NP = 16, 5
MIN_ACTIVE = 2
FLOOR_START = 0.05

M = B * L
N = 2 * H_DIM
TM = 512
TN = 1024
TR = 2048


def _mlp_kernel(x_ref, w1_ref, b1_ref, w2_ref, b2_ref, out_ref):
    n = pl.program_id(1)
    h = x_ref[...] @ w1_ref[...] + b1_ref[...]
    h = 0.5 * h * (1.0 + jax.lax.erf(h * 0.7071067811865476))
    contrib = h @ w2_ref[...]

    @pl.when(n == 0)
    def _():
        out_ref[...] = contrib + b2_ref[...]

    @pl.when(n != 0)
    def _():
        out_ref[...] += contrib


def _epilogue_kernel(z_ref, invt_ref, epsb_ref, out_ref):
    z = z_ref[...] * invt_ref[...]
    zmax = jnp.max(z, axis=-1, keepdims=True)
    e = jnp.exp(z - zmax)
    s = jnp.sum(e, axis=-1, keepdims=True)
    p = e / s
    p_max = jnp.max(p, axis=-1, keepdims=True)
    eps = (FLOOR_START * (1.0 - p_max)) * epsb_ref[...]
    eps_sum = jnp.sum(eps, axis=-1, keepdims=True)
    p = p * (1.0 - eps_sum) + eps
    p = jnp.clip(p, 1e-9, 1.0)
    active = jnp.sum((p > 1e-6).astype(jnp.int32), axis=-1, keepdims=True)
    lane = jax.lax.broadcasted_iota(jnp.int32, p.shape, 1)
    m1 = jnp.max(p, axis=-1, keepdims=True)
    i1 = jnp.min(jnp.where(p == m1, lane, NP), axis=-1, keepdims=True)
    sel1 = lane == i1
    p2 = jnp.where(sel1, -1e30, p)
    m2 = jnp.max(p2, axis=-1, keepdims=True)
    i2 = jnp.min(jnp.where(p2 == m2, lane, NP), axis=-1, keepdims=True)
    mask = sel1 | (lane == i2)
    uniform = mask.astype(jnp.float32) * (1.0 / MIN_ACTIVE)
    out_ref[...] = jnp.where(active < MIN_ACTIVE, uniform, p)


@jax.jit
def kernel(x, W1, b1, W2, b2, gate_log_temp, gate_eps_logit):
    x2 = x.reshape(M, H_DIM)
    logits = pl.pallas_call(
        _mlp_kernel,
        grid=(M // TM, N // TN),
        in_specs=[
            pl.BlockSpec((TM, H_DIM), lambda m, n: (m, 0)),
            pl.BlockSpec((H_DIM, TN), lambda m, n: (0, n)),
            pl.BlockSpec((1, TN), lambda m, n: (0, n)),
            pl.BlockSpec((TN, NH * NP), lambda m, n: (n, 0)),
            pl.BlockSpec((1, NH * NP), lambda m, n: (0, 0)),
        ],
        out_specs=pl.BlockSpec((TM, NH * NP), lambda m, n: (m, 0)),
        out_shape=jax.ShapeDtypeStruct((M, NH * NP), jnp.float32),
        compiler_params=pltpu.CompilerParams(
            dimension_semantics=("parallel", "arbitrary")
        ),
    )(x2, W1, b1.reshape(1, N), W2, b2.reshape(1, NH * NP))

    rows = logits.reshape(M * NH, NP)
    inv_temp = jnp.tile(
        jnp.exp(-gate_log_temp)[:, None], (TR // NH, NP)
    ).reshape(TR, NP)
    eps_base = jnp.tile(jax.nn.sigmoid(gate_eps_logit), (TR // NH, 1))
    probs = pl.pallas_call(
        _epilogue_kernel,
        grid=(M * NH // TR,),
        in_specs=[
            pl.BlockSpec((TR, NP), lambda r: (r, 0)),
            pl.BlockSpec((TR, NP), lambda r: (0, 0)),
            pl.BlockSpec((TR, NP), lambda r: (0, 0)),
        ],
        out_specs=pl.BlockSpec((TR, NP), lambda r: (r, 0)),
        out_shape=jax.ShapeDtypeStruct((M * NH, NP), jnp.float32),
        compiler_params=pltpu.CompilerParams(
            dimension_semantics=("parallel",)
        ),
    )(rows, inv_temp, eps_base)
    return probs.reshape(B, L, NH, NP)
